# baseline (speedup 1.0000x reference)
import jax
import jax.numpy as jnp
from jax.experimental import pallas as pl
from jax.experimental.pallas import tpu as pltpu

_NSTREAM = 8


def _copy_body(perm_ref, *refs):
    o_ref = refs[_NSTREAM]
    sems = refs[_NSTREAM + 1]
    i = pl.program_id(0)

    def copy(s):
        return pltpu.make_async_copy(
            refs[s], o_ref.at[:, pl.ds(_NSTREAM * i + s, 1)], sems.at[s]
        )

    for s in range(_NSTREAM):
        copy(s).start()
    for s in range(_NSTREAM):
        copy(s).wait()


def _in_spec(s):
    return pl.BlockSpec(
        (4, 1, 224, 224), lambda i, perm: (0, perm[_NSTREAM * i + s], 0, 0)
    )


def kernel(x, ldj, permutation):
    B, C, H, W = x.shape
    out = pl.pallas_call(
        _copy_body,
        grid_spec=pltpu.PrefetchScalarGridSpec(
            num_scalar_prefetch=1,
            grid=(C // _NSTREAM,),
            in_specs=[_in_spec(s) for s in range(_NSTREAM)],
            out_specs=pl.BlockSpec(memory_space=pltpu.MemorySpace.HBM),
            scratch_shapes=[pltpu.SemaphoreType.DMA((_NSTREAM,))],
        ),
        out_shape=jax.ShapeDtypeStruct((B, C, H, W), x.dtype),
        compiler_params=pltpu.CompilerParams(
            dimension_semantics=("arbitrary",),
        ),
    )(permutation, *([x] * _NSTREAM))
    return out, ldj


# speedup vs baseline: 1.0511x; 1.0511x over previous
import jax
import jax.numpy as jnp
from jax.experimental import pallas as pl
from jax.experimental.pallas import tpu as pltpu

_NSTREAM = 16


def _copy_body(perm_ref, *refs):
    o_ref = refs[-1]
    for s in range(_NSTREAM):
        o_ref[:, s : s + 1] = refs[s][...]


def _in_spec(s):
    return pl.BlockSpec(
        (4, 1, 224, 224), lambda i, perm: (0, perm[_NSTREAM * i + s], 0, 0)
    )


def kernel(x, ldj, permutation):
    B, C, H, W = x.shape
    out = pl.pallas_call(
        _copy_body,
        grid_spec=pltpu.PrefetchScalarGridSpec(
            num_scalar_prefetch=1,
            grid=(C // _NSTREAM,),
            in_specs=[_in_spec(s) for s in range(_NSTREAM)],
            out_specs=pl.BlockSpec(
                (B, _NSTREAM, H, W), lambda i, perm: (0, i, 0, 0)
            ),
        ),
        out_shape=jax.ShapeDtypeStruct((B, C, H, W), x.dtype),
        compiler_params=pltpu.CompilerParams(
            dimension_semantics=("arbitrary",),
        ),
    )(permutation, *([x] * _NSTREAM))
    return out, ldj
